# baseline (speedup 1.0000x reference)
import functools

import jax
import jax.numpy as jnp
from jax import lax
from jax.experimental import pallas as pl
from jax.experimental.pallas import tpu as pltpu
from jax.experimental.pallas import tpu_sc as plsc

_NC = 2
_NS = 16
_NW = _NC * _NS
_G = 128
_K = 2
_S = _G * _K
_NB = 4


def _lookup_body(table_hbm, idx_hbm, out_hbm, idx_v, bufs,
                 gs0, gs1, gs2, gs3, ss0, ss1, ss2, ss3):
    gsems = (gs0, gs1, gs2, gs3)
    ssems = (ss0, ss1, ss2, ss3)
    wid = lax.axis_index("s") * _NC + lax.axis_index("c")
    bpw = idx_v.shape[0]
    base = wid * bpw
    pltpu.sync_copy(idx_hbm.at[pl.ds(base, bpw)], idx_v)
    nsc = bpw // _S

    def fire(sc, p):
        for k in range(_K):
            off = sc * _S + k * _G
            pltpu.async_copy(
                table_hbm.at[idx_v.at[pl.ds(off, _G)]],
                bufs.at[p].at[pl.ds(k * _G, _G)],
                gsems[p])

    def wait_gather(p):
        pltpu.make_async_copy(
            table_hbm.at[pl.ds(0, _S)], bufs.at[p], gsems[p]).wait()

    def wait_store(p):
        pltpu.make_async_copy(
            bufs.at[p], out_hbm.at[pl.ds(base, _S)], ssems[p]).wait()

    for p in range(_NB - 1):
        fire(p, p)

    @pl.loop(0, nsc // _NB)
    def _(g):
        for p in range(_NB):
            sc = g * _NB + p
            scf = sc + _NB - 1
            pf = (p + _NB - 1) % _NB
            if p == 0:
                @pl.when(g >= 1)
                def _():
                    wait_store(pf)
                fire(scf, pf)
            else:
                wait_store(pf)

                @pl.when(scf < nsc)
                def _():
                    fire(scf, pf)
            wait_gather(p)
            pltpu.async_copy(
                bufs.at[p], out_hbm.at[pl.ds(base + sc * _S, _S)], ssems[p])

    wait_store((nsc - 1) % _NB)


@functools.partial(jax.jit, static_argnames=())
def _gather_rows(idx_flat, table):
    b = idx_flat.shape[0]
    d = table.shape[1]
    bpw = b // _NW
    mesh = plsc.VectorSubcoreMesh(core_axis_name="c", subcore_axis_name="s")
    f = pl.kernel(
        _lookup_body,
        out_type=jax.ShapeDtypeStruct((b, d), jnp.float32),
        mesh=mesh,
        scratch_types=[
            pltpu.VMEM((bpw,), jnp.int32),
            pltpu.VMEM((_NB, _S, d), jnp.float32),
            pltpu.SemaphoreType.DMA,
            pltpu.SemaphoreType.DMA,
            pltpu.SemaphoreType.DMA,
            pltpu.SemaphoreType.DMA,
            pltpu.SemaphoreType.DMA,
            pltpu.SemaphoreType.DMA,
            pltpu.SemaphoreType.DMA,
            pltpu.SemaphoreType.DMA,
        ],
        compiler_params=pltpu.CompilerParams(use_tc_tiling_on_sc=False),
    )
    return f(table, idx_flat)


def _transpose_table_body(in_ref, out_ref):
    out_ref[...] = in_ref[...].T


def _transpose_table(tab_t):
    v = tab_t.shape[1]
    vb = 4096
    return pl.pallas_call(
        _transpose_table_body,
        grid=(pl.cdiv(v, vb),),
        in_specs=[pl.BlockSpec((64, vb), lambda i: (0, i))],
        out_specs=pl.BlockSpec((vb, 64), lambda i: (i, 0)),
        out_shape=jax.ShapeDtypeStruct((v, 64), jnp.float32),
    )(tab_t)


def _transpose_out_body(in_ref, out_ref):
    out_ref[0] = in_ref[0].T


def _transpose_out(g3):
    t, b, d = g3.shape
    return pl.pallas_call(
        _transpose_out_body,
        grid=(t,),
        in_specs=[pl.BlockSpec((1, b, d), lambda i: (i, 0, 0))],
        out_specs=pl.BlockSpec((1, d, b), lambda i: (i, 0, 0)),
        out_shape=jax.ShapeDtypeStruct((t, d, b), jnp.float32),
    )(g3)


def kernel(indices, table):
    nb, nt = indices.shape
    d = table.shape[1]
    tab_r = _transpose_table(table.T)
    idx_flat = indices.T.reshape(nb * nt)
    g = _gather_rows(idx_flat, tab_r)
    out_t = _transpose_out(g.reshape(nt, nb, d))
    return jnp.transpose(out_t, (2, 0, 1))


# speedup vs baseline: 1.4883x; 1.4883x over previous
import functools

import jax
import jax.numpy as jnp
from jax import lax
from jax.experimental import pallas as pl
from jax.experimental.pallas import tpu as pltpu
from jax.experimental.pallas import tpu_sc as plsc

_NC = 2
_NS = 16
_NW = _NC * _NS
_G = 128
_K = 2
_S = _G * _K
_NB = 4

_PB = 2048
_NBLK = 245
_VP = _PB * _NBLK


def _pack_table_body(in_a, in_b, out_ref):
    out_ref[:, 0:64] = in_a[...].T
    out_ref[:, 64:128] = in_b[...].T


def _pack_table(tab_t):
    return pl.pallas_call(
        _pack_table_body,
        grid=(_NBLK,),
        in_specs=[
            pl.BlockSpec((64, _PB), lambda i: (0, i)),
            pl.BlockSpec((64, _PB), lambda i: (0, jnp.minimum(i + _NBLK, 488))),
        ],
        out_specs=pl.BlockSpec((_PB, 128), lambda i: (i, 0)),
        out_shape=jax.ShapeDtypeStruct((_VP, 128), jnp.float32),
    )(tab_t, tab_t)


def _lookup_body(table_hbm, idx_hbm, out_hbm, idx_v, bufs,
                 gs0, gs1, gs2, gs3, ss0, ss1, ss2, ss3):
    gsems = (gs0, gs1, gs2, gs3)
    ssems = (ss0, ss1, ss2, ss3)
    wid = lax.axis_index("s") * _NC + lax.axis_index("c")
    bpw = idx_v.shape[0]
    base = wid * bpw
    pltpu.sync_copy(idx_hbm.at[pl.ds(base, bpw)], idx_v)

    @pl.loop(0, bpw // 16)
    def _(i):
        x = idx_v[pl.ds(i * 16, 16)]
        idx_v[pl.ds(i * 16, 16)] = jnp.where(
            x < _VP, 2 * x, 2 * x - (2 * _VP - 1))

    nsc = bpw // _S

    def fire(sc, p):
        for k in range(_K):
            off = sc * _S + k * _G
            pltpu.async_copy(
                table_hbm.at[idx_v.at[pl.ds(off, _G)]],
                bufs.at[p].at[pl.ds(k * _G, _G)],
                gsems[p])

    def wait_gather(p):
        pltpu.make_async_copy(
            table_hbm.at[pl.ds(0, _S)], bufs.at[p], gsems[p]).wait()

    def wait_store(p):
        pltpu.make_async_copy(
            bufs.at[p], out_hbm.at[pl.ds(base, _S)], ssems[p]).wait()

    for p in range(_NB - 1):
        fire(p, p)

    @pl.loop(0, nsc // _NB)
    def _(g):
        for p in range(_NB):
            sc = g * _NB + p
            scf = sc + _NB - 1
            pf = (p + _NB - 1) % _NB
            if p == 0:
                @pl.when(g >= 1)
                def _():
                    wait_store(pf)
                fire(scf, pf)
            else:
                wait_store(pf)

                @pl.when(scf < nsc)
                def _():
                    fire(scf, pf)
            wait_gather(p)
            pltpu.async_copy(
                bufs.at[p], out_hbm.at[pl.ds(base + sc * _S, _S)], ssems[p])

    wait_store((nsc - 1) % _NB)


@functools.partial(jax.jit, static_argnames=())
def _gather_rows(idx_flat, table):
    b = idx_flat.shape[0]
    d = table.shape[1]
    bpw = b // _NW
    mesh = plsc.VectorSubcoreMesh(core_axis_name="c", subcore_axis_name="s")
    f = pl.kernel(
        _lookup_body,
        out_type=jax.ShapeDtypeStruct((b, d), jnp.float32),
        mesh=mesh,
        scratch_types=[
            pltpu.VMEM((bpw,), jnp.int32),
            pltpu.VMEM((_NB, _S, d), jnp.float32),
            pltpu.SemaphoreType.DMA,
            pltpu.SemaphoreType.DMA,
            pltpu.SemaphoreType.DMA,
            pltpu.SemaphoreType.DMA,
            pltpu.SemaphoreType.DMA,
            pltpu.SemaphoreType.DMA,
            pltpu.SemaphoreType.DMA,
            pltpu.SemaphoreType.DMA,
        ],
        compiler_params=pltpu.CompilerParams(use_tc_tiling_on_sc=False),
    )
    return f(table, idx_flat)


def _unpack_out_body(in_ref, out_ref):
    x = in_ref[...]
    out_ref[0, :, 0:2048] = x[:, 0:64].T
    out_ref[0, :, 2048:4096] = x[:, 64:128].T


def _unpack_out(g2, nt, nb, d):
    rows_per_t = nb * d // 128
    return pl.pallas_call(
        _unpack_out_body,
        grid=(nt,),
        in_specs=[pl.BlockSpec((rows_per_t, 128), lambda i: (i, 0))],
        out_specs=pl.BlockSpec((1, d, nb), lambda i: (i, 0, 0)),
        out_shape=jax.ShapeDtypeStruct((nt, d, nb), jnp.float32),
    )(g2)


def kernel(indices, table):
    nb, nt = indices.shape
    d = table.shape[1]
    tab_p = _pack_table(table.T)
    tab_lin = jnp.reshape(tab_p, (2 * _VP, d))
    idx_perm = (indices.T.reshape(nt, 2, nb // 2)
                .transpose(0, 2, 1).reshape(nb * nt))
    g = _gather_rows(idx_perm, tab_lin)
    g2 = jnp.reshape(g, (nb * nt * d // 128, 128))
    out_t = _unpack_out(g2, nt, nb, d)
    return jnp.transpose(out_t, (2, 0, 1))


# speedup vs baseline: 1.9559x; 1.3142x over previous
import functools

import jax
import jax.numpy as jnp
from jax import lax
from jax.experimental import pallas as pl
from jax.experimental.pallas import tpu as pltpu
from jax.experimental.pallas import tpu_sc as plsc

_NC = 2
_NS = 16
_NW = _NC * _NS
_G = 128
_K = 2
_S = _G * _K
_NB = 4

_PB = 2048
_NBLK = 245
_VP = _PB * _NBLK


def _pack_table_body(in_a, in_b, out_ref):
    out_ref[:, 0:64] = in_a[...].T
    out_ref[:, 64:128] = in_b[...].T


def _pack_table(tab_t):
    return pl.pallas_call(
        _pack_table_body,
        grid=(_NBLK,),
        in_specs=[
            pl.BlockSpec((64, _PB), lambda i: (0, i)),
            pl.BlockSpec((64, _PB), lambda i: (0, jnp.minimum(i + _NBLK, 488))),
        ],
        out_specs=pl.BlockSpec((_PB, 128), lambda i: (i, 0)),
        out_shape=jax.ShapeDtypeStruct((_VP, 128), jnp.float32),
    )(tab_t, tab_t)


def _lookup_body(table_hbm, idx_hbm, out_hbm, idx_v, va, vb, bufs,
                 isem, gs0, gs1, gs2, gs3, ss0, ss1, ss2, ss3):
    gsems = (gs0, gs1, gs2, gs3)
    ssems = (ss0, ss1, ss2, ss3)
    wid = lax.axis_index("s") * _NC + lax.axis_index("c")
    bpw = idx_v.shape[0]
    base = wid * bpw
    npair = bpw // 2
    j0 = wid * npair
    t0 = j0 // 2048
    r0 = j0 - t0 * 2048
    for s in range(7):
        pltpu.async_copy(
            idx_hbm.at[pl.ds((t0 + s) * 4096, 2048)],
            va.at[pl.ds(s * 2048, 2048)], isem)
        pltpu.async_copy(
            idx_hbm.at[pl.ds((t0 + s) * 4096 + 2048, 2048)],
            vb.at[pl.ds(s * 2048, 2048)], isem)
    pltpu.make_async_copy(idx_hbm.at[pl.ds(0, va.shape[0])], va, isem).wait()
    pltpu.make_async_copy(idx_hbm.at[pl.ds(0, vb.shape[0])], vb, isem).wait()

    lane = lax.iota(jnp.int32, 16)

    @pl.loop(0, npair // 16)
    def _(i):
        a = va[pl.ds(r0 + i * 16, 16)]
        b = vb[pl.ds(r0 + i * 16, 16)]
        ra = jnp.where(a < _VP, 2 * a, 2 * a - (2 * _VP - 1))
        rb = jnp.where(b < _VP, 2 * b, 2 * b - (2 * _VP - 1))
        pos = 32 * i + 2 * lane
        plsc.store_scatter(idx_v, [pos], ra)
        plsc.store_scatter(idx_v, [pos + 1], rb)

    nsc = bpw // _S

    def fire(sc, p):
        for k in range(_K):
            off = sc * _S + k * _G
            pltpu.async_copy(
                table_hbm.at[idx_v.at[pl.ds(off, _G)]],
                bufs.at[p].at[pl.ds(k * _G, _G)],
                gsems[p])

    def wait_gather(p):
        pltpu.make_async_copy(
            table_hbm.at[pl.ds(0, _S)], bufs.at[p], gsems[p]).wait()

    def wait_store(p):
        pltpu.make_async_copy(
            bufs.at[p], out_hbm.at[pl.ds(base, _S)], ssems[p]).wait()

    for p in range(_NB - 1):
        fire(p, p)

    @pl.loop(0, nsc // _NB)
    def _(g):
        for p in range(_NB):
            sc = g * _NB + p
            scf = sc + _NB - 1
            pf = (p + _NB - 1) % _NB
            if p == 0:
                @pl.when(g >= 1)
                def _():
                    wait_store(pf)
                fire(scf, pf)
            else:
                wait_store(pf)

                @pl.when(scf < nsc)
                def _():
                    fire(scf, pf)
            wait_gather(p)
            pltpu.async_copy(
                bufs.at[p], out_hbm.at[pl.ds(base + sc * _S, _S)], ssems[p])

    wait_store((nsc - 1) % _NB)


@functools.partial(jax.jit, static_argnames=())
def _gather_rows(idx_flat, table):
    b = idx_flat.shape[0]
    d = table.shape[1]
    bpw = b // _NW
    mesh = plsc.VectorSubcoreMesh(core_axis_name="c", subcore_axis_name="s")
    f = pl.kernel(
        _lookup_body,
        out_type=jax.ShapeDtypeStruct((b, d), jnp.float32),
        mesh=mesh,
        scratch_types=[
            pltpu.VMEM((bpw,), jnp.int32),
            pltpu.VMEM((7 * 2048,), jnp.int32),
            pltpu.VMEM((7 * 2048,), jnp.int32),
            pltpu.VMEM((_NB, _S, d), jnp.float32),
            pltpu.SemaphoreType.DMA,
            pltpu.SemaphoreType.DMA,
            pltpu.SemaphoreType.DMA,
            pltpu.SemaphoreType.DMA,
            pltpu.SemaphoreType.DMA,
            pltpu.SemaphoreType.DMA,
            pltpu.SemaphoreType.DMA,
            pltpu.SemaphoreType.DMA,
            pltpu.SemaphoreType.DMA,
        ],
        compiler_params=pltpu.CompilerParams(
            use_tc_tiling_on_sc=False, needs_layout_passes=False),
    )
    return f(table, idx_flat)


def _unpack_out_body(in_ref, out_ref):
    x = in_ref[...]
    out_ref[0, :, 0:2048] = x[:, 0:64].T
    out_ref[0, :, 2048:4096] = x[:, 64:128].T


def _unpack_out(g2, nt, nb, d):
    rows_per_t = nb * d // 128
    return pl.pallas_call(
        _unpack_out_body,
        grid=(nt,),
        in_specs=[pl.BlockSpec((rows_per_t, 128), lambda i: (i, 0))],
        out_specs=pl.BlockSpec((1, d, nb), lambda i: (i, 0, 0)),
        out_shape=jax.ShapeDtypeStruct((nt, d, nb), jnp.float32),
    )(g2)


def kernel(indices, table):
    nb, nt = indices.shape
    d = table.shape[1]
    tab_p = _pack_table(table.T)
    tab_lin = jnp.reshape(tab_p, (2 * _VP, d))
    idx_flat = indices.T.reshape(nb * nt)
    g = _gather_rows(idx_flat, tab_lin)
    g2 = jnp.reshape(g, (nb * nt * d // 128, 128))
    out_t = _unpack_out(g2, nt, nb, d)
    return jnp.transpose(out_t, (2, 0, 1))


# speedup vs baseline: 2.3284x; 1.1905x over previous
import functools

import jax
import jax.numpy as jnp
from jax import lax
from jax.experimental import pallas as pl
from jax.experimental.pallas import tpu as pltpu
from jax.experimental.pallas import tpu_sc as plsc

_NC = 2
_NS = 16
_NW = _NC * _NS
_G = 128
_K = 2
_S = _G * _K
_NB = 4

_PB = 4096
_NBLK = 123
_VP = _PB * _NBLK
_BCLAMP = 244


def _pack_table_body(in_a, in_b, out_ref):
    out_ref[:, 0:64] = in_a[...].T
    out_ref[:, 64:128] = in_b[...].T


def _pack_table(tab_t):
    return pl.pallas_call(
        _pack_table_body,
        grid=(_NBLK,),
        in_specs=[
            pl.BlockSpec((64, _PB), lambda i: (0, i)),
            pl.BlockSpec((64, _PB),
                         lambda i: (0, jnp.minimum(i + _NBLK, _BCLAMP))),
        ],
        out_specs=pl.BlockSpec((_PB, 128), lambda i: (i, 0)),
        out_shape=jax.ShapeDtypeStruct((_VP, 128), jnp.float32),
    )(tab_t, tab_t)


def _lookup_body(table_hbm, idx_hbm, out_hbm, idx_v, va, vb, bufs,
                 isem, gs0, gs1, gs2, gs3, ss0, ss1, ss2, ss3):
    gsems = (gs0, gs1, gs2, gs3)
    ssems = (ss0, ss1, ss2, ss3)
    wid = lax.axis_index("s") * _NC + lax.axis_index("c")
    bpw = idx_v.shape[0]
    base = wid * bpw
    npair = bpw // 2
    j0 = wid * npair
    t0 = j0 // 2048
    r0 = j0 - t0 * 2048
    for s in range(7):
        pltpu.async_copy(
            idx_hbm.at[pl.ds((t0 + s) * 4096, 2048)],
            va.at[pl.ds(s * 2048, 2048)], isem)
        pltpu.async_copy(
            idx_hbm.at[pl.ds((t0 + s) * 4096 + 2048, 2048)],
            vb.at[pl.ds(s * 2048, 2048)], isem)
    pltpu.make_async_copy(idx_hbm.at[pl.ds(0, va.shape[0])], va, isem).wait()
    pltpu.make_async_copy(idx_hbm.at[pl.ds(0, vb.shape[0])], vb, isem).wait()

    lane = lax.iota(jnp.int32, 16)

    @pl.loop(0, npair // 16)
    def _(i):
        a = va[pl.ds(r0 + i * 16, 16)]
        b = vb[pl.ds(r0 + i * 16, 16)]
        ra = jnp.where(a < _VP, 2 * a, 2 * a - (2 * _VP - 1))
        rb = jnp.where(b < _VP, 2 * b, 2 * b - (2 * _VP - 1))
        pos = 32 * i + 2 * lane
        plsc.store_scatter(idx_v, [pos], ra)
        plsc.store_scatter(idx_v, [pos + 1], rb)

    nsc = bpw // _S

    def fire(sc, p):
        for k in range(_K):
            off = sc * _S + k * _G
            pltpu.async_copy(
                table_hbm.at[idx_v.at[pl.ds(off, _G)]],
                bufs.at[p].at[pl.ds(k * _G, _G)],
                gsems[p])

    def wait_gather(p):
        pltpu.make_async_copy(
            table_hbm.at[pl.ds(0, _S)], bufs.at[p], gsems[p]).wait()

    def wait_store(p):
        pltpu.make_async_copy(
            bufs.at[p], out_hbm.at[pl.ds(base, _S)], ssems[p]).wait()

    for p in range(_NB - 1):
        fire(p, p)

    @pl.loop(0, nsc // _NB)
    def _(g):
        for p in range(_NB):
            sc = g * _NB + p
            scf = sc + _NB - 1
            pf = (p + _NB - 1) % _NB
            if p == 0:
                @pl.when(g >= 1)
                def _():
                    wait_store(pf)
                fire(scf, pf)
            else:
                wait_store(pf)

                @pl.when(scf < nsc)
                def _():
                    fire(scf, pf)
            wait_gather(p)
            pltpu.async_copy(
                bufs.at[p], out_hbm.at[pl.ds(base + sc * _S, _S)], ssems[p])

    wait_store((nsc - 1) % _NB)


@functools.partial(jax.jit, static_argnames=())
def _gather_rows(idx_flat, table):
    b = idx_flat.shape[0]
    d = table.shape[1]
    bpw = b // _NW
    mesh = plsc.VectorSubcoreMesh(core_axis_name="c", subcore_axis_name="s")
    f = pl.kernel(
        _lookup_body,
        out_type=jax.ShapeDtypeStruct((b, d), jnp.float32),
        mesh=mesh,
        scratch_types=[
            pltpu.VMEM((bpw,), jnp.int32),
            pltpu.VMEM((7 * 2048,), jnp.int32),
            pltpu.VMEM((7 * 2048,), jnp.int32),
            pltpu.VMEM((_NB, _S, d), jnp.float32),
            pltpu.SemaphoreType.DMA,
            pltpu.SemaphoreType.DMA,
            pltpu.SemaphoreType.DMA,
            pltpu.SemaphoreType.DMA,
            pltpu.SemaphoreType.DMA,
            pltpu.SemaphoreType.DMA,
            pltpu.SemaphoreType.DMA,
            pltpu.SemaphoreType.DMA,
            pltpu.SemaphoreType.DMA,
        ],
        compiler_params=pltpu.CompilerParams(
            use_tc_tiling_on_sc=False, needs_layout_passes=False),
    )
    return f(table, idx_flat)


def _unpack_out_body(in_ref, out_ref):
    x = in_ref[...]
    out_ref[0, :, 0:2048] = x[0:2048, 0:64].T
    out_ref[0, :, 2048:4096] = x[0:2048, 64:128].T
    out_ref[1, :, 0:2048] = x[2048:4096, 0:64].T
    out_ref[1, :, 2048:4096] = x[2048:4096, 64:128].T


def _unpack_out(g2, nt, nb, d):
    rows_per_t = nb * d // 128
    return pl.pallas_call(
        _unpack_out_body,
        grid=(nt // 2,),
        in_specs=[pl.BlockSpec((2 * rows_per_t, 128), lambda i: (i, 0))],
        out_specs=pl.BlockSpec((2, d, nb), lambda i: (i, 0, 0)),
        out_shape=jax.ShapeDtypeStruct((nt, d, nb), jnp.float32),
    )(g2)


def kernel(indices, table):
    nb, nt = indices.shape
    d = table.shape[1]
    tab_p = _pack_table(table.T)
    tab_lin = jnp.reshape(tab_p, (2 * _VP, d))
    idx_flat = indices.T.reshape(nb * nt)
    g = _gather_rows(idx_flat, tab_lin)
    g2 = jnp.reshape(g, (nb * nt * d // 128, 128))
    out_t = _unpack_out(g2, nt, nb, d)
    return jnp.transpose(out_t, (2, 0, 1))


# speedup vs baseline: 2.5850x; 1.1102x over previous
import functools

import jax
import jax.numpy as jnp
from jax import lax
from jax.experimental import pallas as pl
from jax.experimental.pallas import tpu as pltpu
from jax.experimental.pallas import tpu_sc as plsc

_NC = 2
_NS = 16
_NW = _NC * _NS
_G = 128
_K = 2
_S = _G * _K
_NB = 4

_PB = 8192
_NBLK = 62
_VP = _PB * _NBLK
_BCLAMP = 122


def _pack_table_body(in_a, in_b, out_ref):
    out_ref[:, 0:64] = in_a[...].T
    out_ref[:, 64:128] = in_b[...].T


def _pack_table(tab_t):
    return pl.pallas_call(
        _pack_table_body,
        grid=(_NBLK,),
        in_specs=[
            pl.BlockSpec((64, _PB), lambda i: (0, i)),
            pl.BlockSpec((64, _PB),
                         lambda i: (0, jnp.minimum(i + _NBLK, _BCLAMP))),
        ],
        out_specs=pl.BlockSpec((_PB, 128), lambda i: (i, 0)),
        out_shape=jax.ShapeDtypeStruct((_VP, 128), jnp.float32),
    )(tab_t, tab_t)


def _lookup_body(table_hbm, idx_hbm, out_hbm, idx_v, va, vb, bufs,
                 isem, gs0, gs1, gs2, gs3, ss0, ss1, ss2, ss3):
    gsems = (gs0, gs1, gs2, gs3)
    ssems = (ss0, ss1, ss2, ss3)
    wid = lax.axis_index("s") * _NC + lax.axis_index("c")
    bpw = idx_v.shape[0]
    base = wid * bpw
    npair = bpw // 2
    j0 = wid * npair
    t0 = j0 // 2048
    r0 = j0 - t0 * 2048
    for s in range(7):
        pltpu.async_copy(
            idx_hbm.at[pl.ds((t0 + s) * 4096, 2048)],
            va.at[pl.ds(s * 2048, 2048)], isem)
        pltpu.async_copy(
            idx_hbm.at[pl.ds((t0 + s) * 4096 + 2048, 2048)],
            vb.at[pl.ds(s * 2048, 2048)], isem)
    pltpu.make_async_copy(idx_hbm.at[pl.ds(0, va.shape[0])], va, isem).wait()
    pltpu.make_async_copy(idx_hbm.at[pl.ds(0, vb.shape[0])], vb, isem).wait()

    lane = lax.iota(jnp.int32, 16)

    @pl.loop(0, npair // 16)
    def _(i):
        a = va[pl.ds(r0 + i * 16, 16)]
        b = vb[pl.ds(r0 + i * 16, 16)]
        ra = jnp.where(a < _VP, 2 * a, 2 * a - (2 * _VP - 1))
        rb = jnp.where(b < _VP, 2 * b, 2 * b - (2 * _VP - 1))
        pos = 32 * i + 2 * lane
        plsc.store_scatter(idx_v, [pos], ra)
        plsc.store_scatter(idx_v, [pos + 1], rb)

    nsc = bpw // _S

    def fire(sc, p):
        for k in range(_K):
            off = sc * _S + k * _G
            pltpu.async_copy(
                table_hbm.at[idx_v.at[pl.ds(off, _G)]],
                bufs.at[p].at[pl.ds(k * _G, _G)],
                gsems[p])

    def wait_gather(p):
        pltpu.make_async_copy(
            table_hbm.at[pl.ds(0, _S)], bufs.at[p], gsems[p]).wait()

    def wait_store(p):
        pltpu.make_async_copy(
            bufs.at[p], out_hbm.at[pl.ds(base, _S)], ssems[p]).wait()

    for p in range(_NB - 1):
        fire(p, p)

    @pl.loop(0, nsc // _NB)
    def _(g):
        for p in range(_NB):
            sc = g * _NB + p
            scf = sc + _NB - 1
            pf = (p + _NB - 1) % _NB
            if p == 0:
                @pl.when(g >= 1)
                def _():
                    wait_store(pf)
                fire(scf, pf)
            else:
                wait_store(pf)

                @pl.when(scf < nsc)
                def _():
                    fire(scf, pf)
            wait_gather(p)
            pltpu.async_copy(
                bufs.at[p], out_hbm.at[pl.ds(base + sc * _S, _S)], ssems[p])

    wait_store((nsc - 1) % _NB)


@functools.partial(jax.jit, static_argnames=())
def _gather_rows(idx_flat, table):
    b = idx_flat.shape[0]
    d = table.shape[1]
    bpw = b // _NW
    mesh = plsc.VectorSubcoreMesh(core_axis_name="c", subcore_axis_name="s")
    f = pl.kernel(
        _lookup_body,
        out_type=jax.ShapeDtypeStruct((b, d), jnp.float32),
        mesh=mesh,
        scratch_types=[
            pltpu.VMEM((bpw,), jnp.int32),
            pltpu.VMEM((7 * 2048,), jnp.int32),
            pltpu.VMEM((7 * 2048,), jnp.int32),
            pltpu.VMEM((_NB, _S, d), jnp.float32),
            pltpu.SemaphoreType.DMA,
            pltpu.SemaphoreType.DMA,
            pltpu.SemaphoreType.DMA,
            pltpu.SemaphoreType.DMA,
            pltpu.SemaphoreType.DMA,
            pltpu.SemaphoreType.DMA,
            pltpu.SemaphoreType.DMA,
            pltpu.SemaphoreType.DMA,
            pltpu.SemaphoreType.DMA,
        ],
        compiler_params=pltpu.CompilerParams(
            use_tc_tiling_on_sc=False, needs_layout_passes=False),
    )
    return f(table, idx_flat)


def _unpack_out_body(in_ref, out_ref):
    x = in_ref[...]
    for t in range(4):
        out_ref[t, :, 0:2048] = x[t * 2048:(t + 1) * 2048, 0:64].T
        out_ref[t, :, 2048:4096] = x[t * 2048:(t + 1) * 2048, 64:128].T


def _unpack_out(g2, nt, nb, d):
    rows_per_t = nb * d // 128
    return pl.pallas_call(
        _unpack_out_body,
        grid=(nt // 4,),
        in_specs=[pl.BlockSpec((4 * rows_per_t, 128), lambda i: (i, 0))],
        out_specs=pl.BlockSpec((4, d, nb), lambda i: (i, 0, 0)),
        out_shape=jax.ShapeDtypeStruct((nt, d, nb), jnp.float32),
    )(g2)


def kernel(indices, table):
    nb, nt = indices.shape
    d = table.shape[1]
    tab_p = _pack_table(table.T)
    tab_lin = jnp.reshape(tab_p, (2 * _VP, d))
    idx_flat = indices.T.reshape(nb * nt)
    g = _gather_rows(idx_flat, tab_lin)
    g2 = jnp.reshape(g, (nb * nt * d // 128, 128))
    out_t = _unpack_out(g2, nt, nb, d)
    return jnp.transpose(out_t, (2, 0, 1))


# speedup vs baseline: 2.7053x; 1.0465x over previous
import functools

import jax
import jax.numpy as jnp
from jax import lax
from jax.experimental import pallas as pl
from jax.experimental.pallas import tpu as pltpu
from jax.experimental.pallas import tpu_sc as plsc

_NC = 2
_NS = 16
_NW = _NC * _NS
_G = 128
_K = 2
_S = _G * _K
_NB = 4

_PB = 16384
_NBLK = 31
_VP = _PB * _NBLK
_BCLAMP = 61


def _pack_table_body(in_a, in_b, out_ref):
    out_ref[:, 0:64] = in_a[...].T
    out_ref[:, 64:128] = in_b[...].T


def _pack_table(tab_t):
    return pl.pallas_call(
        _pack_table_body,
        grid=(_NBLK,),
        in_specs=[
            pl.BlockSpec((64, _PB), lambda i: (0, i)),
            pl.BlockSpec((64, _PB),
                         lambda i: (0, jnp.minimum(i + _NBLK, _BCLAMP))),
        ],
        out_specs=pl.BlockSpec((_PB, 128), lambda i: (i, 0)),
        out_shape=jax.ShapeDtypeStruct((_VP, 128), jnp.float32),
    )(tab_t, tab_t)


def _lookup_body(table_hbm, idx_hbm, out_hbm, idx_v, va, vb, bufs,
                 isem, gs0, gs1, gs2, gs3, ss0, ss1, ss2, ss3):
    gsems = (gs0, gs1, gs2, gs3)
    ssems = (ss0, ss1, ss2, ss3)
    wid = lax.axis_index("s") * _NC + lax.axis_index("c")
    bpw = idx_v.shape[0]
    base = wid * bpw
    npair = bpw // 2
    j0 = wid * npair
    t0 = j0 // 2048
    r0 = j0 - t0 * 2048
    for s in range(7):
        pltpu.async_copy(
            idx_hbm.at[pl.ds((t0 + s) * 4096, 2048)],
            va.at[pl.ds(s * 2048, 2048)], isem)
        pltpu.async_copy(
            idx_hbm.at[pl.ds((t0 + s) * 4096 + 2048, 2048)],
            vb.at[pl.ds(s * 2048, 2048)], isem)
    pltpu.make_async_copy(idx_hbm.at[pl.ds(0, va.shape[0])], va, isem).wait()
    pltpu.make_async_copy(idx_hbm.at[pl.ds(0, vb.shape[0])], vb, isem).wait()

    lane = lax.iota(jnp.int32, 16)

    @pl.loop(0, npair // 16)
    def _(i):
        a = va[pl.ds(r0 + i * 16, 16)]
        b = vb[pl.ds(r0 + i * 16, 16)]
        ra = jnp.where(a < _VP, 2 * a, 2 * a - (2 * _VP - 1))
        rb = jnp.where(b < _VP, 2 * b, 2 * b - (2 * _VP - 1))
        pos = 32 * i + 2 * lane
        plsc.store_scatter(idx_v, [pos], ra)
        plsc.store_scatter(idx_v, [pos + 1], rb)

    nsc = bpw // _S

    def fire(sc, p):
        for k in range(_K):
            off = sc * _S + k * _G
            pltpu.async_copy(
                table_hbm.at[idx_v.at[pl.ds(off, _G)]],
                bufs.at[p].at[pl.ds(k * _G, _G)],
                gsems[p])

    def wait_gather(p):
        pltpu.make_async_copy(
            table_hbm.at[pl.ds(0, _S)], bufs.at[p], gsems[p]).wait()

    def wait_store(p):
        pltpu.make_async_copy(
            bufs.at[p], out_hbm.at[pl.ds(base, _S)], ssems[p]).wait()

    for p in range(_NB - 1):
        fire(p, p)

    @pl.loop(0, nsc // _NB)
    def _(g):
        for p in range(_NB):
            sc = g * _NB + p
            scf = sc + _NB - 1
            pf = (p + _NB - 1) % _NB
            if p == 0:
                @pl.when(g >= 1)
                def _():
                    wait_store(pf)
                fire(scf, pf)
            else:
                wait_store(pf)

                @pl.when(scf < nsc)
                def _():
                    fire(scf, pf)
            wait_gather(p)
            pltpu.async_copy(
                bufs.at[p], out_hbm.at[pl.ds(base + sc * _S, _S)], ssems[p])

    wait_store((nsc - 1) % _NB)


@functools.partial(jax.jit, static_argnames=())
def _gather_rows(idx_flat, table):
    b = idx_flat.shape[0]
    d = table.shape[1]
    bpw = b // _NW
    mesh = plsc.VectorSubcoreMesh(core_axis_name="c", subcore_axis_name="s")
    f = pl.kernel(
        _lookup_body,
        out_type=jax.ShapeDtypeStruct((b, d), jnp.float32),
        mesh=mesh,
        scratch_types=[
            pltpu.VMEM((bpw,), jnp.int32),
            pltpu.VMEM((7 * 2048,), jnp.int32),
            pltpu.VMEM((7 * 2048,), jnp.int32),
            pltpu.VMEM((_NB, _S, d), jnp.float32),
            pltpu.SemaphoreType.DMA,
            pltpu.SemaphoreType.DMA,
            pltpu.SemaphoreType.DMA,
            pltpu.SemaphoreType.DMA,
            pltpu.SemaphoreType.DMA,
            pltpu.SemaphoreType.DMA,
            pltpu.SemaphoreType.DMA,
            pltpu.SemaphoreType.DMA,
            pltpu.SemaphoreType.DMA,
        ],
        compiler_params=pltpu.CompilerParams(
            use_tc_tiling_on_sc=False, needs_layout_passes=False),
    )
    return f(table, idx_flat)


def _unpack_out_body(in_ref, out_ref):
    x = in_ref[...]
    for t in range(8):
        out_ref[t, :, 0:2048] = x[t * 2048:(t + 1) * 2048, 0:64].T
        out_ref[t, :, 2048:4096] = x[t * 2048:(t + 1) * 2048, 64:128].T


def _unpack_out(g2, nt, nb, d):
    rows_per_t = nb * d // 128
    return pl.pallas_call(
        _unpack_out_body,
        grid=(nt // 8,),
        in_specs=[pl.BlockSpec((8 * rows_per_t, 128), lambda i: (i, 0))],
        out_specs=pl.BlockSpec((8, d, nb), lambda i: (i, 0, 0)),
        out_shape=jax.ShapeDtypeStruct((nt, d, nb), jnp.float32),
    )(g2)


def kernel(indices, table):
    nb, nt = indices.shape
    d = table.shape[1]
    tab_p = _pack_table(table.T)
    tab_lin = jnp.reshape(tab_p, (2 * _VP, d))
    idx_flat = indices.T.reshape(nb * nt)
    g = _gather_rows(idx_flat, tab_lin)
    g2 = jnp.reshape(g, (nb * nt * d // 128, 128))
    out_t = _unpack_out(g2, nt, nb, d)
    return jnp.transpose(out_t, (2, 0, 1))


# speedup vs baseline: 3.2126x; 1.1875x over previous
import functools

import jax
import jax.numpy as jnp
from jax import lax
from jax.experimental import pallas as pl
from jax.experimental.pallas import tpu as pltpu
from jax.experimental.pallas import tpu_sc as plsc

_NC = 2
_NS = 16
_NW = _NC * _NS
_G = 128
_K = 2
_S = _G * _K
_NB = 4

_PB = 16384
_NBLK = 31
_VP = _PB * _NBLK
_BCLAMP = 61


def _pack_table_body(in_a, in_b, out_ref):
    out_ref[...] = jnp.concatenate([in_a[...], in_b[...]], axis=0).T


def _pack_table(tab_t):
    return pl.pallas_call(
        _pack_table_body,
        grid=(_NBLK,),
        in_specs=[
            pl.BlockSpec((64, _PB), lambda i: (0, i)),
            pl.BlockSpec((64, _PB),
                         lambda i: (0, jnp.minimum(i + _NBLK, _BCLAMP))),
        ],
        out_specs=pl.BlockSpec((_PB, 128), lambda i: (i, 0)),
        out_shape=jax.ShapeDtypeStruct((_VP, 128), jnp.float32),
    )(tab_t, tab_t)


def _lookup_body(table_hbm, idx_hbm, out_hbm, idx_v, va, vb, bufs,
                 isem, gs0, gs1, gs2, gs3, ss0, ss1, ss2, ss3):
    gsems = (gs0, gs1, gs2, gs3)
    ssems = (ss0, ss1, ss2, ss3)
    wid = lax.axis_index("s") * _NC + lax.axis_index("c")
    bpw = idx_v.shape[0]
    base = wid * bpw
    npair = bpw // 2
    j0 = wid * npair
    t0 = j0 // 2048
    r0 = j0 - t0 * 2048
    for s in range(7):
        pltpu.async_copy(
            idx_hbm.at[pl.ds((t0 + s) * 4096, 2048)],
            va.at[pl.ds(s * 2048, 2048)], isem)
        pltpu.async_copy(
            idx_hbm.at[pl.ds((t0 + s) * 4096 + 2048, 2048)],
            vb.at[pl.ds(s * 2048, 2048)], isem)
    pltpu.make_async_copy(idx_hbm.at[pl.ds(0, va.shape[0])], va, isem).wait()
    pltpu.make_async_copy(idx_hbm.at[pl.ds(0, vb.shape[0])], vb, isem).wait()

    lane = lax.iota(jnp.int32, 16)

    @pl.loop(0, npair // 16)
    def _(i):
        a = va[pl.ds(r0 + i * 16, 16)]
        b = vb[pl.ds(r0 + i * 16, 16)]
        ra = jnp.where(a < _VP, 2 * a, 2 * a - (2 * _VP - 1))
        rb = jnp.where(b < _VP, 2 * b, 2 * b - (2 * _VP - 1))
        pos = 32 * i + 2 * lane
        plsc.store_scatter(idx_v, [pos], ra)
        plsc.store_scatter(idx_v, [pos + 1], rb)

    nsc = bpw // _S

    def fire(sc, p):
        for k in range(_K):
            off = sc * _S + k * _G
            pltpu.async_copy(
                table_hbm.at[idx_v.at[pl.ds(off, _G)]],
                bufs.at[p].at[pl.ds(k * _G, _G)],
                gsems[p])

    def wait_gather(p):
        pltpu.make_async_copy(
            table_hbm.at[pl.ds(0, _S)], bufs.at[p], gsems[p]).wait()

    def wait_store(p):
        pltpu.make_async_copy(
            bufs.at[p], out_hbm.at[pl.ds(base, _S)], ssems[p]).wait()

    for p in range(_NB - 1):
        fire(p, p)

    @pl.loop(0, nsc // _NB)
    def _(g):
        for p in range(_NB):
            sc = g * _NB + p
            scf = sc + _NB - 1
            pf = (p + _NB - 1) % _NB
            if p == 0:
                @pl.when(g >= 1)
                def _():
                    wait_store(pf)
                fire(scf, pf)
            else:
                wait_store(pf)

                @pl.when(scf < nsc)
                def _():
                    fire(scf, pf)
            wait_gather(p)
            pltpu.async_copy(
                bufs.at[p], out_hbm.at[pl.ds(base + sc * _S, _S)], ssems[p])

    wait_store((nsc - 1) % _NB)


@functools.partial(jax.jit, static_argnames=())
def _gather_rows(idx_flat, table):
    b = idx_flat.shape[0]
    d = table.shape[1]
    bpw = b // _NW
    mesh = plsc.VectorSubcoreMesh(core_axis_name="c", subcore_axis_name="s")
    f = pl.kernel(
        _lookup_body,
        out_type=jax.ShapeDtypeStruct((b, d), jnp.float32),
        mesh=mesh,
        scratch_types=[
            pltpu.VMEM((bpw,), jnp.int32),
            pltpu.VMEM((7 * 2048,), jnp.int32),
            pltpu.VMEM((7 * 2048,), jnp.int32),
            pltpu.VMEM((_NB, _S, d), jnp.float32),
            pltpu.SemaphoreType.DMA,
            pltpu.SemaphoreType.DMA,
            pltpu.SemaphoreType.DMA,
            pltpu.SemaphoreType.DMA,
            pltpu.SemaphoreType.DMA,
            pltpu.SemaphoreType.DMA,
            pltpu.SemaphoreType.DMA,
            pltpu.SemaphoreType.DMA,
            pltpu.SemaphoreType.DMA,
        ],
        compiler_params=pltpu.CompilerParams(
            use_tc_tiling_on_sc=False, needs_layout_passes=False),
    )
    return f(table, idx_flat)


def _unpack_out_body(in_ref, out_ref):
    for t in range(8):
        y = in_ref[t * 2048:(t + 1) * 2048, :].T
        out_ref[t, :, 0:2048] = y[0:64, :]
        out_ref[t, :, 2048:4096] = y[64:128, :]


def _unpack_out(g2, nt, nb, d):
    rows_per_t = nb * d // 128
    return pl.pallas_call(
        _unpack_out_body,
        grid=(nt // 8,),
        in_specs=[pl.BlockSpec((8 * rows_per_t, 128), lambda i: (i, 0))],
        out_specs=pl.BlockSpec((8, d, nb), lambda i: (i, 0, 0)),
        out_shape=jax.ShapeDtypeStruct((nt, d, nb), jnp.float32),
    )(g2)


def kernel(indices, table):
    nb, nt = indices.shape
    d = table.shape[1]
    tab_p = _pack_table(table.T)
    tab_lin = jnp.reshape(tab_p, (2 * _VP, d))
    idx_flat = indices.T.reshape(nb * nt)
    g = _gather_rows(idx_flat, tab_lin)
    g2 = jnp.reshape(g, (nb * nt * d // 128, 128))
    out_t = _unpack_out(g2, nt, nb, d)
    return jnp.transpose(out_t, (2, 0, 1))
